# baseline (speedup 1.0000x reference)
import functools

import jax
import jax.numpy as jnp
from jax import lax
from jax.experimental import pallas as pl
from jax.experimental.pallas import tpu as pltpu
from jax.experimental.pallas import tpu_sc as plsc

EMB = 64
HOM = 3
_CHUNK = 64


@functools.lru_cache(maxsize=None)
def _make_sc_gather(NSLAB, B):
    CH = 256
    b_per_w = B // 2
    n_chunks = b_per_w // CH
    n_pairs = n_chunks // 2
    assert n_chunks * CH == b_per_w and n_pairs * 2 == n_chunks
    mesh = plsc.ScalarSubcoreMesh(axis_name="c", num_cores=2)

    @functools.partial(
        pl.kernel,
        mesh=mesh,
        out_type=jax.ShapeDtypeStruct((B, EMB), jnp.float32),
        scratch_types=[
            pltpu.SMEM((CH,), jnp.int32),
            pltpu.SMEM((CH,), jnp.int32),
            pltpu.SemaphoreType.DMA,
            pltpu.SemaphoreType.DMA,
            pltpu.SemaphoreType.DMA,
        ],
        compiler_params=pltpu.CompilerParams(use_tc_tiling_on_sc=True),
    )
    def gather_k(table_hbm, idxr_hbm, out_hbm, idx_s0, idx_s1, sem_rows, semi0, semi1):
        cid = lax.axis_index("c")
        base = cid * b_per_w
        semis = [semi0, semi1]
        idx_bufs = [idx_s0, idx_s1]

        def load_idx(ch, buf):
            pltpu.async_copy(
                idxr_hbm.at[pl.ds(base + ch * CH, CH)], idx_bufs[buf], semis[buf]
            )

        def wait_idx(buf):
            pltpu.make_async_copy(
                idxr_hbm.at[pl.ds(0, CH)], idx_bufs[buf], semis[buf]
            ).wait()

        def fire_rows(ch, buf):
            def row(i, _):
                r = idx_bufs[buf][i]
                pltpu.async_copy(
                    table_hbm.at[r >> 3, r & 7],
                    out_hbm.at[base + ch * CH + i],
                    sem_rows,
                )
                return 0

            lax.fori_loop(0, CH, row, 0, unroll=8)

        load_idx(0, 0)
        load_idx(1, 1)

        def body(p, _):
            ch = 2 * p
            wait_idx(0)
            fire_rows(ch, 0)
            load_idx(ch + 2, 0)
            wait_idx(1)
            fire_rows(ch + 1, 1)
            load_idx(ch + 3, 1)
            return 0

        lax.fori_loop(0, n_pairs - 1, body, 0, unroll=False)
        wait_idx(0)
        fire_rows(n_chunks - 2, 0)
        wait_idx(1)
        fire_rows(n_chunks - 1, 1)
        pltpu.make_async_copy(
            out_hbm.at[pl.ds(0, b_per_w)],
            out_hbm.at[pl.ds(base, b_per_w)],
            sem_rows,
        ).wait()

    return gather_k


def _dense_body(c_ref, d_ref, wt_ref, b_ref, out_ref):
    c = c_ref[...]
    tgt = jax.nn.sigmoid(d_ref[...])
    acc = None
    for k in range(HOM):
        est = jax.nn.sigmoid(
            jnp.dot(c, wt_ref[k], preferred_element_type=jnp.float32) + b_ref[k]
        )
        diff = est - tgt
        dist = jnp.sqrt(jnp.sum(diff * diff, axis=1, keepdims=True) + 1e-12)
        acc = dist if acc is None else acc + dist
    out_ref[...] = acc * (1.0 / HOM)


@functools.lru_cache(maxsize=None)
def _make_tc_dense(B1, BB):
    nb = B1 // BB
    assert nb * BB == B1
    return pl.pallas_call(
        _dense_body,
        grid=(nb,),
        in_specs=[
            pl.BlockSpec((BB, EMB), lambda g: (g, 0)),
            pl.BlockSpec((BB, EMB), lambda g: (g + nb, 0)),
            pl.BlockSpec((HOM, EMB, EMB), lambda g: (0, 0, 0)),
            pl.BlockSpec((HOM, 1, EMB), lambda g: (0, 0, 0)),
        ],
        out_specs=pl.BlockSpec((BB, 1), lambda g: (g, 0)),
        out_shape=jax.ShapeDtypeStruct((B1, 1), jnp.float32),
    )


def kernel(data, idx, embed, embed_rel, hom_W, hom_b):
    B1 = data.shape[0]
    V, D = embed.shape
    table3 = embed.reshape(V // 8, 8, D)
    idx_all = jnp.concatenate([data[:, 0], data[:, 1]])
    cd = _make_sc_gather(V // 8, 2 * B1)(table3, idx_all)
    wt = jnp.transpose(hom_W, (0, 2, 1))
    b3 = hom_b[:, None, :]
    loss = _make_tc_dense(B1, 512)(cd, cd, wt, b3)[:, 0]
    guard = jnp.where(jnp.asarray(idx) != 0, jnp.float32(jnp.nan), jnp.float32(0.0))
    return loss + guard


# speedup vs baseline: 1.1223x; 1.1223x over previous
import functools

import jax
import jax.numpy as jnp
from jax import lax
from jax.experimental import pallas as pl
from jax.experimental.pallas import tpu as pltpu
from jax.experimental.pallas import tpu_sc as plsc

EMB = 64
HOM = 3
_IDX_W = 128
_CHUNK = 512


@functools.lru_cache(maxsize=None)
def _make_sc_gather(V2, D2, B):
    info = plsc.get_sparse_core_info()
    NW = info.num_cores * info.num_subcores
    NC = info.num_cores
    b_per_w = B // NW
    n_chunks = b_per_w // _CHUNK
    streams_per_chunk = _CHUNK // _IDX_W
    idx_rows_per_w = b_per_w // _IDX_W
    assert b_per_w * NW == B and n_chunks * _CHUNK == b_per_w
    mesh = plsc.VectorSubcoreMesh(core_axis_name="c", subcore_axis_name="s")

    @functools.partial(
        pl.kernel,
        mesh=mesh,
        out_type=jax.ShapeDtypeStruct((B, D2), jnp.float32),
        scratch_types=[
            pltpu.VMEM((idx_rows_per_w, _IDX_W), jnp.int32),
            pltpu.VMEM((_CHUNK, D2), jnp.float32),
            pltpu.SemaphoreType.DMA,
        ],
        compiler_params=pltpu.CompilerParams(use_tc_tiling_on_sc=True),
    )
    def gather_k(table_hbm, idx_hbm, out_hbm, idx_v, rows_v, sem):
        wid = lax.axis_index("s") * NC + lax.axis_index("c")
        base = wid * b_per_w
        pltpu.sync_copy(idx_hbm.at[pl.ds(wid * idx_rows_per_w, idx_rows_per_w)], idx_v)
        for ch in range(n_chunks):
            copies = []
            for j in range(streams_per_chunk):
                copies.append(
                    pltpu.async_copy(
                        table_hbm.at[idx_v.at[ch * streams_per_chunk + j]],
                        rows_v.at[pl.ds(j * _IDX_W, _IDX_W)],
                        sem,
                    )
                )
            for cp in copies:
                cp.wait()
            pltpu.sync_copy(rows_v, out_hbm.at[pl.ds(base + ch * _CHUNK, _CHUNK)])

    return gather_k


def _dense_body(c_ref, d_ref, pc_ref, pd_ref, wt_ref, b_ref, out_ref):
    c2 = c_ref[...]
    d2 = d_ref[...]
    pc = pc_ref[...] != 0
    pd = pd_ref[...] != 0
    c = jnp.where(pc, c2[:, EMB:], c2[:, :EMB])
    d = jnp.where(pd, d2[:, EMB:], d2[:, :EMB])
    tgt = jax.nn.sigmoid(d)
    acc = None
    for k in range(HOM):
        est = jax.nn.sigmoid(
            jnp.dot(c, wt_ref[k], preferred_element_type=jnp.float32) + b_ref[k]
        )
        diff = est - tgt
        dist = jnp.sqrt(jnp.sum(diff * diff, axis=1, keepdims=True) + 1e-12)
        acc = dist if acc is None else acc + dist
    out_ref[...] = acc * (1.0 / HOM)


@functools.lru_cache(maxsize=None)
def _make_tc_dense(B1, BB):
    nb = B1 // BB
    assert nb * BB == B1
    return pl.pallas_call(
        _dense_body,
        grid=(nb,),
        in_specs=[
            pl.BlockSpec((BB, 2 * EMB), lambda g: (g, 0)),
            pl.BlockSpec((BB, 2 * EMB), lambda g: (g + nb, 0)),
            pl.BlockSpec((BB, 1), lambda g: (g, 0)),
            pl.BlockSpec((BB, 1), lambda g: (g, 0)),
            pl.BlockSpec((HOM, EMB, EMB), lambda g: (0, 0, 0)),
            pl.BlockSpec((HOM, 1, EMB), lambda g: (0, 0, 0)),
        ],
        out_specs=pl.BlockSpec((BB, 1), lambda g: (g, 0)),
        out_shape=jax.ShapeDtypeStruct((B1, 1), jnp.float32),
    )


def kernel(data, idx, embed, embed_rel, hom_W, hom_b):
    B1 = data.shape[0]
    V, D = embed.shape
    table2 = embed.reshape(V // 2, 2 * D)
    idx_all = jnp.concatenate([data[:, 0], data[:, 1]])
    idx2 = (idx_all >> 1).reshape(-1, _IDX_W)
    cd = _make_sc_gather(V // 2, 2 * D, 2 * B1)(table2, idx2)
    pc = (data[:, 0] & 1).reshape(B1, 1)
    pd = (data[:, 1] & 1).reshape(B1, 1)
    wt = jnp.transpose(hom_W, (0, 2, 1))
    b3 = hom_b[:, None, :]
    loss = _make_tc_dense(B1, 512)(cd, cd, pc, pd, wt, b3)[:, 0]
    guard = jnp.where(jnp.asarray(idx) != 0, jnp.float32(jnp.nan), jnp.float32(0.0))
    return loss + guard
